# baseline (speedup 1.0000x reference)
import functools

import jax
import jax.numpy as jnp
from jax import lax
from jax.experimental import pallas as pl
from jax.experimental.pallas import tpu as pltpu
from jax.experimental.pallas import tpu_sc as plsc

N_NODES = 10000
N_EDGES = 320000
D_FEAT = 128
HID = 8
DP = 16
DH = D_FEAT // 2
SLOPE = 0.01

NC, NS = 2, 16
NW = NC * NS
CHUNK = 128
NCHUNK = 80
EPW = NCHUNK * CHUNK
E_PAD = NW * EPW
N_PAD = 10240
ACC_STRIPE = N_PAD // NS
TAB_STRIPE = N_NODES // NS


def _leaky(v):
    return jnp.where(v >= 0, v, SLOPE * v)


def _ring_loop(nbuf, tab_sh, acc_sh, src_v, dst_v, rows, gsem, ssem):
    ngroup = NCHUNK // nbuf

    def group(jj, carry):
        base = jj * nbuf
        gs = [pltpu.async_copy(tab_sh.at[src_v.at[base + b]], rows[b], gsem[b])
              for b in range(nbuf)]
        ss = []
        for b in range(nbuf):
            gs[b].wait()
            ss.append(pltpu.async_copy(
                rows[b], acc_sh.at[dst_v.at[base + b]], ssem[b], add=True))
        for b in range(nbuf):
            ss[b].wait()
        return carry

    lax.fori_loop(0, ngroup, group, 0)


def _sc_segsum_wide(xl, xr, src3, dst3, zeros):
    mesh = plsc.VectorSubcoreMesh(core_axis_name="c", subcore_axis_name="s")
    NBUF = 2

    @functools.partial(
        pl.kernel,
        out_type=jax.ShapeDtypeStruct((2, NC, N_PAD, DH), jnp.float32),
        mesh=mesh,
        compiler_params=pltpu.CompilerParams(use_tc_tiling_on_sc=False),
        scratch_types=[
            pltpu.VMEM((NCHUNK, CHUNK), jnp.int32),
            pltpu.VMEM((NCHUNK, CHUNK), jnp.int32),
            [pltpu.VMEM((CHUNK, DH), jnp.float32) for _ in range(NBUF)],
            pltpu.VMEM_SHARED((N_NODES, DH), jnp.float32),
            pltpu.VMEM_SHARED((N_PAD, DH), jnp.float32),
            [pltpu.SemaphoreType.DMA for _ in range(NBUF)],
            [pltpu.SemaphoreType.DMA for _ in range(NBUF)],
        ],
    )
    def seg_kernel(xl_hbm, xr_hbm, src_hbm, dst_hbm, zero_hbm, out_hbm,
                   src_v, dst_v, rows, tab_sh, acc_sh, gsem, ssem):
        cid = lax.axis_index("c")
        sid = lax.axis_index("s")
        wid = sid * NC + cid
        tstr = pl.ds(sid * TAB_STRIPE, TAB_STRIPE)
        astr = pl.ds(sid * ACC_STRIPE, ACC_STRIPE)

        pltpu.sync_copy(src_hbm.at[wid], src_v)
        pltpu.sync_copy(dst_hbm.at[wid], dst_v)
        pltpu.sync_copy(xl_hbm.at[tstr], tab_sh.at[tstr])
        pltpu.sync_copy(zero_hbm.at[astr], acc_sh.at[astr])
        plsc.subcore_barrier()

        _ring_loop(NBUF, tab_sh, acc_sh, src_v, dst_v, rows, gsem, ssem)

        plsc.subcore_barrier()
        pltpu.sync_copy(acc_sh.at[astr], out_hbm.at[0, cid, astr])
        pltpu.sync_copy(zero_hbm.at[astr], acc_sh.at[astr])
        pltpu.sync_copy(xr_hbm.at[tstr], tab_sh.at[tstr])
        plsc.subcore_barrier()

        _ring_loop(NBUF, tab_sh, acc_sh, src_v, dst_v, rows, gsem, ssem)

        plsc.subcore_barrier()
        pltpu.sync_copy(acc_sh.at[astr], out_hbm.at[1, cid, astr])

    return seg_kernel(xl, xr, src3, dst3, zeros)


def _sc_segsum_hid(h, src3, dst3, zeros):
    mesh = plsc.VectorSubcoreMesh(core_axis_name="c", subcore_axis_name="s")
    NBUF = 4

    @functools.partial(
        pl.kernel,
        out_type=jax.ShapeDtypeStruct((NC, N_PAD, DP), jnp.float32),
        mesh=mesh,
        compiler_params=pltpu.CompilerParams(use_tc_tiling_on_sc=False),
        scratch_types=[
            pltpu.VMEM((NCHUNK, CHUNK), jnp.int32),
            pltpu.VMEM((NCHUNK, CHUNK), jnp.int32),
            [pltpu.VMEM((CHUNK, DP), jnp.float32) for _ in range(NBUF)],
            pltpu.VMEM_SHARED((N_NODES, DP), jnp.float32),
            pltpu.VMEM_SHARED((N_PAD, DP), jnp.float32),
            [pltpu.SemaphoreType.DMA for _ in range(NBUF)],
            [pltpu.SemaphoreType.DMA for _ in range(NBUF)],
        ],
    )
    def seg_kernel(h_hbm, src_hbm, dst_hbm, zero_hbm, out_hbm,
                   src_v, dst_v, rows, tab_sh, acc_sh, gsem, ssem):
        cid = lax.axis_index("c")
        sid = lax.axis_index("s")
        wid = sid * NC + cid
        tstr = pl.ds(sid * TAB_STRIPE, TAB_STRIPE)
        astr = pl.ds(sid * ACC_STRIPE, ACC_STRIPE)

        pltpu.sync_copy(src_hbm.at[wid], src_v)
        pltpu.sync_copy(dst_hbm.at[wid], dst_v)
        pltpu.sync_copy(h_hbm.at[tstr], tab_sh.at[tstr])
        pltpu.sync_copy(zero_hbm.at[astr], acc_sh.at[astr])
        plsc.subcore_barrier()

        _ring_loop(NBUF, tab_sh, acc_sh, src_v, dst_v, rows, gsem, ssem)

        plsc.subcore_barrier()
        pltpu.sync_copy(acc_sh.at[astr], out_hbm.at[cid, astr])

    return seg_kernel(h, src3, dst3, zeros)


def _tc_layer0(x, seg, w1p, b1p, w2p, b2p):
    def body(x_ref, s_ref, w1_ref, b1_ref, w2_ref, b2_ref, o_ref):
        agg = jnp.concatenate(
            [s_ref[0, 0, :N_NODES] + s_ref[0, 1, :N_NODES],
             s_ref[1, 0, :N_NODES] + s_ref[1, 1, :N_NODES]], axis=1)
        hp = x_ref[...] + agg
        a = _leaky(jnp.dot(hp, w1_ref[...],
                           preferred_element_type=jnp.float32) + b1_ref[...])
        r = jnp.dot(a, w2_ref[...],
                    preferred_element_type=jnp.float32) + b2_ref[...]
        o_ref[...] = _leaky(r)
    return pl.pallas_call(
        body,
        out_shape=jax.ShapeDtypeStruct((N_NODES, DP), jnp.float32),
    )(x, seg, w1p, b1p, w2p, b2p)


def _tc_layer(h, seg, w1p, b1p, w2p, b2p, last):
    def body(h_ref, s_ref, w1_ref, b1_ref, w2_ref, b2_ref, o_ref):
        hp = h_ref[...] + (s_ref[0, :N_NODES] + s_ref[1, :N_NODES])
        a = _leaky(jnp.dot(hp, w1_ref[...],
                           preferred_element_type=jnp.float32) + b1_ref[...])
        r = jnp.dot(a, w2_ref[...],
                    preferred_element_type=jnp.float32) + b2_ref[...]
        o_ref[...] = r if last else _leaky(r)
    return pl.pallas_call(
        body,
        out_shape=jax.ShapeDtypeStruct((N_NODES, DP), jnp.float32),
    )(h, seg, w1p, b1p, w2p, b2p)


def _tc_head(h4, f1p, f1b, f2, f2b):
    def body(h_ref, f1_ref, f1b_ref, f2_ref, f2b_ref, o_ref):
        g = _leaky(h_ref[...])
        t = jnp.sum(g * f1_ref[...], axis=1, keepdims=True) + f1b_ref[0, 0]
        z = _leaky(t)
        u = jnp.sum(z * f2_ref[...], axis=0, keepdims=True) + f2b_ref[...]
        m = jnp.max(u, axis=1, keepdims=True)
        lse = m + jnp.log(jnp.sum(jnp.exp(u - m), axis=1, keepdims=True))
        o_ref[...] = u - lse
    return pl.pallas_call(
        body,
        out_shape=jax.ShapeDtypeStruct((1, 2), jnp.float32),
    )(h4, f1p, f1b, f2, f2b)


def _pad_mat(w, rows, cols):
    return jnp.zeros((rows, cols), jnp.float32).at[:w.shape[0], :w.shape[1]].set(w)


def _pad_row(b, cols):
    return jnp.zeros((1, cols), jnp.float32).at[0, :b.shape[0]].set(b)


def kernel(x, W1_0, b1_0, W2_0, b2_0, W1_1, b1_1, W2_1, b2_1,
           W1_2, b1_2, W2_2, b2_2, W1_3, b1_3, W2_3, b2_3,
           FC1_W, FC1_b, FC2_W, FC2_b, edge_index, batch):
    pad_n = E_PAD - N_EDGES
    pad_idx = jnp.arange(pad_n, dtype=jnp.int32) % 16
    src3 = jnp.concatenate([edge_index[0], pad_idx]).reshape(NW, NCHUNK, CHUNK)
    dst3 = jnp.concatenate([edge_index[1], N_NODES + pad_idx]).reshape(NW, NCHUNK, CHUNK)
    xl = x[:, :DH]
    xr = x[:, DH:]
    zeros_wide = jnp.zeros((N_PAD, DH), jnp.float32)
    zeros_hid = jnp.zeros((N_PAD, DP), jnp.float32)

    w1p = [_pad_mat(W1_0, D_FEAT, DP)] + \
          [_pad_mat(w, DP, DP) for w in (W1_1, W1_2, W1_3)]
    b1p = [_pad_row(b, DP) for b in (b1_0, b1_1, b1_2, b1_3)]
    w2p = [_pad_mat(w, DP, DP) for w in (W2_0, W2_1, W2_2, W2_3)]
    b2p = [_pad_row(b, DP) for b in (b2_0, b2_1, b2_2, b2_3)]
    f1p = _pad_row(FC1_W[:, 0], DP)
    f1b = FC1_b.reshape(1, 1)
    f2b = FC2_b.reshape(1, 2)

    seg = _sc_segsum_wide(xl, xr, src3, dst3, zeros_wide)
    h = _tc_layer0(x, seg, w1p[0], b1p[0], w2p[0], b2p[0])
    for l in (1, 2, 3):
        seg = _sc_segsum_hid(h, src3, dst3, zeros_hid)
        h = _tc_layer(h, seg, w1p[l], b1p[l], w2p[l], b2p[l], last=(l == 3))
    out = _tc_head(h, f1p, f1b, FC2_W, f2b)
    return out[0]


# speedup vs baseline: 1.0807x; 1.0807x over previous
import functools

import jax
import jax.numpy as jnp
from jax import lax
from jax.experimental import pallas as pl
from jax.experimental.pallas import tpu as pltpu
from jax.experimental.pallas import tpu_sc as plsc

N_NODES = 10000
N_EDGES = 320000
D_FEAT = 128
HID = 8
DP = 16
DH = D_FEAT // 2
SLOPE = 0.01

NC, NS = 2, 16
NW = NC * NS
CHUNK = 128
NCHUNK = 80
EPW = NCHUNK * CHUNK
E_PAD = NW * EPW
N_PAD = 10240
ACC_STRIPE = N_PAD // NS
TAB_STRIPE = N_NODES // NS


def _leaky(v):
    return jnp.where(v >= 0, v, SLOPE * v)


def _ring_loop(nbuf, tab_sh, acc_sh, src_v, dst_v, rows, gsem, ssem):
    ngroup = NCHUNK // nbuf

    def group(jj, carry):
        base = jj * nbuf
        gs = [pltpu.async_copy(tab_sh.at[src_v.at[base + b]], rows[b], gsem[b])
              for b in range(nbuf)]
        ss = []
        for b in range(nbuf):
            gs[b].wait()
            ss.append(pltpu.async_copy(
                rows[b], acc_sh.at[dst_v.at[base + b]], ssem[b], add=True))
        for b in range(nbuf):
            ss[b].wait()
        return carry

    lax.fori_loop(0, ngroup, group, 0)


def _sc_segsum_wide(xl, xr, src3, dst3, zeros):
    mesh = plsc.VectorSubcoreMesh(core_axis_name="c", subcore_axis_name="s")
    NBUF = 4

    @functools.partial(
        pl.kernel,
        out_type=jax.ShapeDtypeStruct((2, NC, N_PAD, DH), jnp.float32),
        mesh=mesh,
        compiler_params=pltpu.CompilerParams(use_tc_tiling_on_sc=False),
        scratch_types=[
            pltpu.VMEM((NCHUNK, CHUNK), jnp.int32),
            pltpu.VMEM((NCHUNK, CHUNK), jnp.int32),
            [pltpu.VMEM((CHUNK, DH), jnp.float32) for _ in range(NBUF)],
            pltpu.VMEM_SHARED((N_PAD, DH), jnp.float32),
            [pltpu.SemaphoreType.DMA for _ in range(NBUF)],
            [pltpu.SemaphoreType.DMA for _ in range(NBUF)],
        ],
    )
    def seg_kernel(xl_hbm, xr_hbm, src_hbm, dst_hbm, zero_hbm, out_hbm,
                   src_v, dst_v, rows, acc_sh, gsem, ssem):
        cid = lax.axis_index("c")
        sid = lax.axis_index("s")
        wid = sid * NC + cid
        astr = pl.ds(sid * ACC_STRIPE, ACC_STRIPE)

        pltpu.sync_copy(src_hbm.at[wid], src_v)
        pltpu.sync_copy(dst_hbm.at[wid], dst_v)
        pltpu.sync_copy(zero_hbm.at[astr], acc_sh.at[astr])
        plsc.subcore_barrier()

        _ring_loop(NBUF, xl_hbm, acc_sh, src_v, dst_v, rows, gsem, ssem)

        plsc.subcore_barrier()
        pltpu.sync_copy(acc_sh.at[astr], out_hbm.at[0, cid, astr])
        pltpu.sync_copy(zero_hbm.at[astr], acc_sh.at[astr])
        plsc.subcore_barrier()

        _ring_loop(NBUF, xr_hbm, acc_sh, src_v, dst_v, rows, gsem, ssem)

        plsc.subcore_barrier()
        pltpu.sync_copy(acc_sh.at[astr], out_hbm.at[1, cid, astr])

    return seg_kernel(xl, xr, src3, dst3, zeros)


def _sc_segsum_hid(h, src3, dst3, zeros):
    mesh = plsc.VectorSubcoreMesh(core_axis_name="c", subcore_axis_name="s")
    NBUF = 4

    @functools.partial(
        pl.kernel,
        out_type=jax.ShapeDtypeStruct((NC, N_PAD, DP), jnp.float32),
        mesh=mesh,
        compiler_params=pltpu.CompilerParams(use_tc_tiling_on_sc=False),
        scratch_types=[
            pltpu.VMEM((NCHUNK, CHUNK), jnp.int32),
            pltpu.VMEM((NCHUNK, CHUNK), jnp.int32),
            [pltpu.VMEM((CHUNK, DP), jnp.float32) for _ in range(NBUF)],
            pltpu.VMEM_SHARED((N_NODES, DP), jnp.float32),
            pltpu.VMEM_SHARED((N_PAD, DP), jnp.float32),
            [pltpu.SemaphoreType.DMA for _ in range(NBUF)],
            [pltpu.SemaphoreType.DMA for _ in range(NBUF)],
        ],
    )
    def seg_kernel(h_hbm, src_hbm, dst_hbm, zero_hbm, out_hbm,
                   src_v, dst_v, rows, tab_sh, acc_sh, gsem, ssem):
        cid = lax.axis_index("c")
        sid = lax.axis_index("s")
        wid = sid * NC + cid
        tstr = pl.ds(sid * TAB_STRIPE, TAB_STRIPE)
        astr = pl.ds(sid * ACC_STRIPE, ACC_STRIPE)

        pltpu.sync_copy(src_hbm.at[wid], src_v)
        pltpu.sync_copy(dst_hbm.at[wid], dst_v)
        pltpu.sync_copy(h_hbm.at[tstr], tab_sh.at[tstr])
        pltpu.sync_copy(zero_hbm.at[astr], acc_sh.at[astr])
        plsc.subcore_barrier()

        _ring_loop(NBUF, tab_sh, acc_sh, src_v, dst_v, rows, gsem, ssem)

        plsc.subcore_barrier()
        pltpu.sync_copy(acc_sh.at[astr], out_hbm.at[cid, astr])

    return seg_kernel(h, src3, dst3, zeros)


def _tc_layer0(x, seg, w1p, b1p, w2p, b2p):
    def body(x_ref, s_ref, w1_ref, b1_ref, w2_ref, b2_ref, o_ref):
        agg = jnp.concatenate(
            [s_ref[0, 0, :N_NODES] + s_ref[0, 1, :N_NODES],
             s_ref[1, 0, :N_NODES] + s_ref[1, 1, :N_NODES]], axis=1)
        hp = x_ref[...] + agg
        a = _leaky(jnp.dot(hp, w1_ref[...],
                           preferred_element_type=jnp.float32) + b1_ref[...])
        r = jnp.dot(a, w2_ref[...],
                    preferred_element_type=jnp.float32) + b2_ref[...]
        o_ref[...] = _leaky(r)
    return pl.pallas_call(
        body,
        out_shape=jax.ShapeDtypeStruct((N_NODES, DP), jnp.float32),
    )(x, seg, w1p, b1p, w2p, b2p)


def _tc_layer(h, seg, w1p, b1p, w2p, b2p, last):
    def body(h_ref, s_ref, w1_ref, b1_ref, w2_ref, b2_ref, o_ref):
        hp = h_ref[...] + (s_ref[0, :N_NODES] + s_ref[1, :N_NODES])
        a = _leaky(jnp.dot(hp, w1_ref[...],
                           preferred_element_type=jnp.float32) + b1_ref[...])
        r = jnp.dot(a, w2_ref[...],
                    preferred_element_type=jnp.float32) + b2_ref[...]
        o_ref[...] = r if last else _leaky(r)
    return pl.pallas_call(
        body,
        out_shape=jax.ShapeDtypeStruct((N_NODES, DP), jnp.float32),
    )(h, seg, w1p, b1p, w2p, b2p)


def _tc_head(h4, f1p, f1b, f2, f2b):
    def body(h_ref, f1_ref, f1b_ref, f2_ref, f2b_ref, o_ref):
        g = _leaky(h_ref[...])
        t = jnp.sum(g * f1_ref[...], axis=1, keepdims=True) + f1b_ref[0, 0]
        z = _leaky(t)
        u = jnp.sum(z * f2_ref[...], axis=0, keepdims=True) + f2b_ref[...]
        m = jnp.max(u, axis=1, keepdims=True)
        lse = m + jnp.log(jnp.sum(jnp.exp(u - m), axis=1, keepdims=True))
        o_ref[...] = u - lse
    return pl.pallas_call(
        body,
        out_shape=jax.ShapeDtypeStruct((1, 2), jnp.float32),
    )(h4, f1p, f1b, f2, f2b)


def _pad_mat(w, rows, cols):
    return jnp.zeros((rows, cols), jnp.float32).at[:w.shape[0], :w.shape[1]].set(w)


def _pad_row(b, cols):
    return jnp.zeros((1, cols), jnp.float32).at[0, :b.shape[0]].set(b)


def kernel(x, W1_0, b1_0, W2_0, b2_0, W1_1, b1_1, W2_1, b2_1,
           W1_2, b1_2, W2_2, b2_2, W1_3, b1_3, W2_3, b2_3,
           FC1_W, FC1_b, FC2_W, FC2_b, edge_index, batch):
    pad_n = E_PAD - N_EDGES
    pad_idx = jnp.arange(pad_n, dtype=jnp.int32) % 16
    src3 = jnp.concatenate([edge_index[0], pad_idx]).reshape(NW, NCHUNK, CHUNK)
    dst3 = jnp.concatenate([edge_index[1], N_NODES + pad_idx]).reshape(NW, NCHUNK, CHUNK)
    xl = x[:, :DH]
    xr = x[:, DH:]
    zeros_wide = jnp.zeros((N_PAD, DH), jnp.float32)
    zeros_hid = jnp.zeros((N_PAD, DP), jnp.float32)

    w1p = [_pad_mat(W1_0, D_FEAT, DP)] + \
          [_pad_mat(w, DP, DP) for w in (W1_1, W1_2, W1_3)]
    b1p = [_pad_row(b, DP) for b in (b1_0, b1_1, b1_2, b1_3)]
    w2p = [_pad_mat(w, DP, DP) for w in (W2_0, W2_1, W2_2, W2_3)]
    b2p = [_pad_row(b, DP) for b in (b2_0, b2_1, b2_2, b2_3)]
    f1p = _pad_row(FC1_W[:, 0], DP)
    f1b = FC1_b.reshape(1, 1)
    f2b = FC2_b.reshape(1, 2)

    seg = _sc_segsum_wide(xl, xr, src3, dst3, zeros_wide)
    h = _tc_layer0(x, seg, w1p[0], b1p[0], w2p[0], b2p[0])
    for l in (1, 2, 3):
        seg = _sc_segsum_hid(h, src3, dst3, zeros_hid)
        h = _tc_layer(h, seg, w1p[l], b1p[l], w2p[l], b2p[l], last=(l == 3))
    out = _tc_head(h, f1p, f1b, FC2_W, f2b)
    return out[0]


# speedup vs baseline: 1.2051x; 1.1151x over previous
import functools

import jax
import jax.numpy as jnp
from jax import lax
from jax.experimental import pallas as pl
from jax.experimental.pallas import tpu as pltpu
from jax.experimental.pallas import tpu_sc as plsc

N_NODES = 10000
N_EDGES = 320000
D_FEAT = 128
HID = 8
DP = 16
DH = D_FEAT // 2
SLOPE = 0.01

NC, NS = 2, 16
NW = NC * NS
CHUNK = 128
NCHUNK = 80
EPW = NCHUNK * CHUNK
E_PAD = NW * EPW
N_PAD = 10240
ACC_STRIPE = N_PAD // NS
TAB_STRIPE = N_NODES // NS


def _leaky(v):
    return jnp.where(v >= 0, v, SLOPE * v)


def _ring_loop(nbuf, tab_sh, acc_sh, src_v, dst_v, rows, gsem, ssem):
    ngroup = NCHUNK // nbuf

    def group(jj, carry):
        base = jj * nbuf
        gs = [pltpu.async_copy(tab_sh.at[src_v.at[base + b]], rows[b], gsem[b])
              for b in range(nbuf)]
        ss = []
        for b in range(nbuf):
            gs[b].wait()
            ss.append(pltpu.async_copy(
                rows[b], acc_sh.at[dst_v.at[base + b]], ssem[b], add=True))
        for b in range(nbuf):
            ss[b].wait()
        return carry

    lax.fori_loop(0, ngroup, group, 0)


def _sc_segsum_wide(xl, xr, src3, dst3, zeros):
    mesh = plsc.VectorSubcoreMesh(core_axis_name="c", subcore_axis_name="s")
    NBUF = 8

    @functools.partial(
        pl.kernel,
        out_type=jax.ShapeDtypeStruct((2, NC, N_PAD, DH), jnp.float32),
        mesh=mesh,
        compiler_params=pltpu.CompilerParams(use_tc_tiling_on_sc=False),
        scratch_types=[
            pltpu.VMEM((NCHUNK, CHUNK), jnp.int32),
            pltpu.VMEM((NCHUNK, CHUNK), jnp.int32),
            [pltpu.VMEM((CHUNK, DH), jnp.float32) for _ in range(NBUF)],
            pltpu.VMEM_SHARED((N_PAD, DH), jnp.float32),
            [pltpu.SemaphoreType.DMA for _ in range(NBUF)],
            [pltpu.SemaphoreType.DMA for _ in range(NBUF)],
        ],
    )
    def seg_kernel(xl_hbm, xr_hbm, src_hbm, dst_hbm, zero_hbm, out_hbm,
                   src_v, dst_v, rows, acc_sh, gsem, ssem):
        cid = lax.axis_index("c")
        sid = lax.axis_index("s")
        wid = sid * NC + cid
        astr = pl.ds(sid * ACC_STRIPE, ACC_STRIPE)

        pltpu.sync_copy(src_hbm.at[wid], src_v)
        pltpu.sync_copy(dst_hbm.at[wid], dst_v)
        pltpu.sync_copy(zero_hbm.at[astr], acc_sh.at[astr])
        plsc.subcore_barrier()

        _ring_loop(NBUF, xl_hbm, acc_sh, src_v, dst_v, rows, gsem, ssem)

        plsc.subcore_barrier()
        pltpu.sync_copy(acc_sh.at[astr], out_hbm.at[0, cid, astr])
        pltpu.sync_copy(zero_hbm.at[astr], acc_sh.at[astr])
        plsc.subcore_barrier()

        _ring_loop(NBUF, xr_hbm, acc_sh, src_v, dst_v, rows, gsem, ssem)

        plsc.subcore_barrier()
        pltpu.sync_copy(acc_sh.at[astr], out_hbm.at[1, cid, astr])

    return seg_kernel(xl, xr, src3, dst3, zeros)


def _sc_segsum_hid(h, src3, dst3, zeros):
    mesh = plsc.VectorSubcoreMesh(core_axis_name="c", subcore_axis_name="s")
    NBUF = 8

    @functools.partial(
        pl.kernel,
        out_type=jax.ShapeDtypeStruct((NC, N_PAD, DP), jnp.float32),
        mesh=mesh,
        compiler_params=pltpu.CompilerParams(use_tc_tiling_on_sc=False),
        scratch_types=[
            pltpu.VMEM((NCHUNK, CHUNK), jnp.int32),
            pltpu.VMEM((NCHUNK, CHUNK), jnp.int32),
            [pltpu.VMEM((CHUNK, DP), jnp.float32) for _ in range(NBUF)],
            pltpu.VMEM_SHARED((N_NODES, DP), jnp.float32),
            pltpu.VMEM_SHARED((N_PAD, DP), jnp.float32),
            [pltpu.SemaphoreType.DMA for _ in range(NBUF)],
            [pltpu.SemaphoreType.DMA for _ in range(NBUF)],
        ],
    )
    def seg_kernel(h_hbm, src_hbm, dst_hbm, zero_hbm, out_hbm,
                   src_v, dst_v, rows, tab_sh, acc_sh, gsem, ssem):
        cid = lax.axis_index("c")
        sid = lax.axis_index("s")
        wid = sid * NC + cid
        tstr = pl.ds(sid * TAB_STRIPE, TAB_STRIPE)
        astr = pl.ds(sid * ACC_STRIPE, ACC_STRIPE)

        pltpu.sync_copy(src_hbm.at[wid], src_v)
        pltpu.sync_copy(dst_hbm.at[wid], dst_v)
        pltpu.sync_copy(h_hbm.at[tstr], tab_sh.at[tstr])
        pltpu.sync_copy(zero_hbm.at[astr], acc_sh.at[astr])
        plsc.subcore_barrier()

        _ring_loop(NBUF, tab_sh, acc_sh, src_v, dst_v, rows, gsem, ssem)

        plsc.subcore_barrier()
        pltpu.sync_copy(acc_sh.at[astr], out_hbm.at[cid, astr])

    return seg_kernel(h, src3, dst3, zeros)


def _tc_layer0(x, seg, w1p, b1p, w2p, b2p):
    def body(x_ref, s_ref, w1_ref, b1_ref, w2_ref, b2_ref, o_ref):
        agg = jnp.concatenate(
            [s_ref[0, 0, :N_NODES] + s_ref[0, 1, :N_NODES],
             s_ref[1, 0, :N_NODES] + s_ref[1, 1, :N_NODES]], axis=1)
        hp = x_ref[...] + agg
        a = _leaky(jnp.dot(hp, w1_ref[...],
                           preferred_element_type=jnp.float32) + b1_ref[...])
        r = jnp.dot(a, w2_ref[...],
                    preferred_element_type=jnp.float32) + b2_ref[...]
        o_ref[...] = _leaky(r)
    return pl.pallas_call(
        body,
        out_shape=jax.ShapeDtypeStruct((N_NODES, DP), jnp.float32),
    )(x, seg, w1p, b1p, w2p, b2p)


def _tc_layer(h, seg, w1p, b1p, w2p, b2p, last):
    def body(h_ref, s_ref, w1_ref, b1_ref, w2_ref, b2_ref, o_ref):
        hp = h_ref[...] + (s_ref[0, :N_NODES] + s_ref[1, :N_NODES])
        a = _leaky(jnp.dot(hp, w1_ref[...],
                           preferred_element_type=jnp.float32) + b1_ref[...])
        r = jnp.dot(a, w2_ref[...],
                    preferred_element_type=jnp.float32) + b2_ref[...]
        o_ref[...] = r if last else _leaky(r)
    return pl.pallas_call(
        body,
        out_shape=jax.ShapeDtypeStruct((N_NODES, DP), jnp.float32),
    )(h, seg, w1p, b1p, w2p, b2p)


def _tc_head(h4, f1p, f1b, f2, f2b):
    def body(h_ref, f1_ref, f1b_ref, f2_ref, f2b_ref, o_ref):
        g = _leaky(h_ref[...])
        t = jnp.sum(g * f1_ref[...], axis=1, keepdims=True) + f1b_ref[0, 0]
        z = _leaky(t)
        u = jnp.sum(z * f2_ref[...], axis=0, keepdims=True) + f2b_ref[...]
        m = jnp.max(u, axis=1, keepdims=True)
        lse = m + jnp.log(jnp.sum(jnp.exp(u - m), axis=1, keepdims=True))
        o_ref[...] = u - lse
    return pl.pallas_call(
        body,
        out_shape=jax.ShapeDtypeStruct((1, 2), jnp.float32),
    )(h4, f1p, f1b, f2, f2b)


def _pad_mat(w, rows, cols):
    return jnp.zeros((rows, cols), jnp.float32).at[:w.shape[0], :w.shape[1]].set(w)


def _pad_row(b, cols):
    return jnp.zeros((1, cols), jnp.float32).at[0, :b.shape[0]].set(b)


def kernel(x, W1_0, b1_0, W2_0, b2_0, W1_1, b1_1, W2_1, b2_1,
           W1_2, b1_2, W2_2, b2_2, W1_3, b1_3, W2_3, b2_3,
           FC1_W, FC1_b, FC2_W, FC2_b, edge_index, batch):
    pad_n = E_PAD - N_EDGES
    pad_idx = jnp.arange(pad_n, dtype=jnp.int32) % 16
    src3 = jnp.concatenate([edge_index[0], pad_idx]).reshape(NW, NCHUNK, CHUNK)
    dst3 = jnp.concatenate([edge_index[1], N_NODES + pad_idx]).reshape(NW, NCHUNK, CHUNK)
    xl = x[:, :DH]
    xr = x[:, DH:]
    zeros_wide = jnp.zeros((N_PAD, DH), jnp.float32)
    zeros_hid = jnp.zeros((N_PAD, DP), jnp.float32)

    w1p = [_pad_mat(W1_0, D_FEAT, DP)] + \
          [_pad_mat(w, DP, DP) for w in (W1_1, W1_2, W1_3)]
    b1p = [_pad_row(b, DP) for b in (b1_0, b1_1, b1_2, b1_3)]
    w2p = [_pad_mat(w, DP, DP) for w in (W2_0, W2_1, W2_2, W2_3)]
    b2p = [_pad_row(b, DP) for b in (b2_0, b2_1, b2_2, b2_3)]
    f1p = _pad_row(FC1_W[:, 0], DP)
    f1b = FC1_b.reshape(1, 1)
    f2b = FC2_b.reshape(1, 2)

    seg = _sc_segsum_wide(xl, xr, src3, dst3, zeros_wide)
    h = _tc_layer0(x, seg, w1p[0], b1p[0], w2p[0], b2p[0])
    for l in (1, 2, 3):
        seg = _sc_segsum_hid(h, src3, dst3, zeros_hid)
        h = _tc_layer(h, seg, w1p[l], b1p[l], w2p[l], b2p[l], last=(l == 3))
    out = _tc_head(h, f1p, f1b, FC2_W, f2b)
    return out[0]
